# baseline (speedup 1.0000x reference)
import jax
import jax.numpy as jnp
from jax import lax
from jax.experimental import pallas as pl
from jax.experimental.pallas import tpu as pltpu
from jax.experimental.pallas import tpu_sc as plsc

N = 100000
SEG = 512
LANES = 16
NS = 16
NV_TOTAL = N // LANES
NV_LO = NV_TOTAL // NS
EXTRA = NV_TOTAL - NV_LO * NS
CHUNK_LO = NV_LO * LANES
CHUNK_HI = (NV_LO + 1) * LANES
ACC_W = 2 * SEG
ROW_STRIDE = ACC_W + 1
ACC_FLAT = 65 * 256


def _dice_body(pred_hbm, target_hbm, batch_hbm, out_hbm,
               pred_v, target_v, batch_v, acc, partial_v, stage, out_v,
               shared, sem_p, sem_t, sem_b):
    wid = lax.axis_index("s")
    has_extra = wid < EXTRA
    base = wid * CHUNK_LO + jnp.minimum(wid, EXTRA) * LANES
    nv = NV_LO + has_extra.astype(jnp.int32)

    @pl.when(has_extra)
    def _():
        pltpu.async_copy(pred_hbm.at[pl.ds(base, CHUNK_HI)], pred_v, sem_p)
        pltpu.async_copy(target_hbm.at[pl.ds(base, CHUNK_HI)], target_v, sem_t)
        pltpu.async_copy(batch_hbm.at[pl.ds(base, CHUNK_HI)], batch_v, sem_b)

    @pl.when(jnp.logical_not(has_extra))
    def _():
        pltpu.async_copy(pred_hbm.at[pl.ds(base, CHUNK_LO)],
                         pred_v.at[pl.ds(0, CHUNK_LO)], sem_p)
        pltpu.async_copy(target_hbm.at[pl.ds(base, CHUNK_LO)],
                         target_v.at[pl.ds(0, CHUNK_LO)], sem_t)
        pltpu.async_copy(batch_hbm.at[pl.ds(base, CHUNK_LO)],
                         batch_v.at[pl.ds(0, CHUNK_LO)], sem_b)

    zero = jnp.zeros((LANES,), jnp.float32)

    def zero_body(cb, carry):
        off = cb * (16 * LANES)
        for r in range(16):
            acc[pl.ds(off + r * LANES, LANES)] = zero
        return carry

    lax.fori_loop(0, ACC_FLAT // (16 * LANES), zero_body, 0)

    @pl.when(has_extra)
    def _():
        pltpu.make_async_copy(pred_hbm.at[pl.ds(base, CHUNK_HI)], pred_v, sem_p).wait()
        pltpu.make_async_copy(target_hbm.at[pl.ds(base, CHUNK_HI)], target_v, sem_t).wait()
        pltpu.make_async_copy(batch_hbm.at[pl.ds(base, CHUNK_HI)], batch_v, sem_b).wait()

    @pl.when(jnp.logical_not(has_extra))
    def _():
        pltpu.make_async_copy(pred_hbm.at[pl.ds(base, CHUNK_LO)],
                              pred_v.at[pl.ds(0, CHUNK_LO)], sem_p).wait()
        pltpu.make_async_copy(target_hbm.at[pl.ds(base, CHUNK_LO)],
                              target_v.at[pl.ds(0, CHUNK_LO)], sem_t).wait()
        pltpu.make_async_copy(batch_hbm.at[pl.ds(base, CHUNK_LO)],
                              batch_v.at[pl.ds(0, CHUNK_LO)], sem_b).wait()

    row_off = lax.iota(jnp.int32, LANES) * ROW_STRIDE

    def scat_one(off):
        p = pred_v[pl.ds(off, LANES)]
        t = target_v[pl.ds(off, LANES)]
        b = batch_v[pl.ds(off, LANES)]
        idx = row_off + b
        plsc.addupdate_scatter(acc, [idx], p * t)
        plsc.addupdate_scatter(acc, [idx + SEG], p + t)

    UNROLL = 5

    def body(j, carry):
        for u in range(UNROLL):
            scat_one((j * UNROLL + u) * LANES)
        return carry

    lax.fori_loop(0, NV_LO // UNROLL, body, 0)

    @pl.when(has_extra)
    def _():
        scat_one(NV_LO * LANES)

    def red_body(cb, carry):
        off = cb * LANES
        v = acc[pl.ds(off, LANES)]
        for r in range(1, LANES):
            v = v + acc[pl.ds(r * ROW_STRIDE + off, LANES)]
        partial_v[pl.ds(off, LANES)] = v
        return carry

    lax.fori_loop(0, ACC_W // LANES, red_body, 0)

    pltpu.sync_copy(partial_v, shared.at[wid])
    plsc.subcore_barrier()

    @pl.when(wid == 0)
    def _():
        pltpu.sync_copy(shared, stage)

        def red2_body(cb, carry):
            off = cb * LANES
            v = stage[0, pl.ds(off, LANES)]
            for r in range(1, LANES):
                v = v + stage[r, pl.ds(off, LANES)]
            partial_v[pl.ds(off, LANES)] = v
            return carry

        lax.fori_loop(0, ACC_W // LANES, red2_body, 0)

        def dice_body(j, s_acc):
            off = j * LANES
            iv = partial_v[pl.ds(off, LANES)]
            dv = partial_v[pl.ds(SEG + off, LANES)]
            return s_acc + (2.0 * iv + 1.0) / (dv + 1.0)

        dice_sum = lax.fori_loop(0, SEG // LANES, dice_body,
                                 jnp.zeros((LANES,), jnp.float32))
        total = jnp.sum(dice_sum)
        out_v[...] = jnp.broadcast_to(float(SEG) - total, (LANES,))
        pltpu.sync_copy(out_v, out_hbm)


_dice_sc = pl.kernel(
    _dice_body,
    out_type=jax.ShapeDtypeStruct((LANES,), jnp.float32),
    mesh=plsc.VectorSubcoreMesh(core_axis_name="c", subcore_axis_name="s",
                                num_cores=1, num_subcores=NS),
    scratch_types=[
        pltpu.VMEM((CHUNK_HI,), jnp.float32),
        pltpu.VMEM((CHUNK_HI,), jnp.float32),
        pltpu.VMEM((CHUNK_HI,), jnp.int32),
        pltpu.VMEM((ACC_FLAT,), jnp.float32),
        pltpu.VMEM((ACC_W,), jnp.float32),
        pltpu.VMEM((LANES, ACC_W), jnp.float32),
        pltpu.VMEM((LANES,), jnp.float32),
        pltpu.VMEM_SHARED((LANES, ACC_W), jnp.float32),
        pltpu.SemaphoreType.DMA,
        pltpu.SemaphoreType.DMA,
        pltpu.SemaphoreType.DMA,
    ],
    compiler_params=pltpu.CompilerParams(needs_layout_passes=False),
)


def kernel(pred, target, batch):
    return _dice_sc(pred, target, batch.astype(jnp.int32))[0]


# speedup vs baseline: 1.0677x; 1.0677x over previous
import jax
import jax.numpy as jnp
from jax import lax
from jax.experimental import pallas as pl
from jax.experimental.pallas import tpu as pltpu
from jax.experimental.pallas import tpu_sc as plsc

N = 100000
SEG = 512
LANES = 16
NS = 16
NV_TOTAL = N // LANES
NV_LO = NV_TOTAL // NS
EXTRA = NV_TOTAL - NV_LO * NS
CHUNK_LO = NV_LO * LANES
CHUNK_HI = (NV_LO + 1) * LANES
ACC_W = 2 * SEG
ROW_STRIDE = ACC_W + 1
ACC_FLAT = 65 * 256


def _dice_body(pred_hbm, target_hbm, batch_hbm, out_hbm,
               pred_v, target_v, batch_v, acc, partial_v, stage, out_v,
               shared, sem_p, sem_t, sem_b):
    wid = lax.axis_index("s")
    has_extra = wid < EXTRA
    base = wid * CHUNK_LO + jnp.minimum(wid, EXTRA) * LANES
    nv = NV_LO + has_extra.astype(jnp.int32)

    @pl.when(has_extra)
    def _():
        pltpu.async_copy(pred_hbm.at[pl.ds(base, CHUNK_HI)], pred_v, sem_p)
        pltpu.async_copy(target_hbm.at[pl.ds(base, CHUNK_HI)], target_v, sem_t)
        pltpu.async_copy(batch_hbm.at[pl.ds(base, CHUNK_HI)], batch_v, sem_b)

    @pl.when(jnp.logical_not(has_extra))
    def _():
        pltpu.async_copy(pred_hbm.at[pl.ds(base, CHUNK_LO)],
                         pred_v.at[pl.ds(0, CHUNK_LO)], sem_p)
        pltpu.async_copy(target_hbm.at[pl.ds(base, CHUNK_LO)],
                         target_v.at[pl.ds(0, CHUNK_LO)], sem_t)
        pltpu.async_copy(batch_hbm.at[pl.ds(base, CHUNK_LO)],
                         batch_v.at[pl.ds(0, CHUNK_LO)], sem_b)

    zero = jnp.zeros((LANES,), jnp.float32)

    def zero_body(cb, carry):
        off = cb * (16 * LANES)
        for r in range(16):
            acc[pl.ds(off + r * LANES, LANES)] = zero
        return carry

    lax.fori_loop(0, ACC_FLAT // (16 * LANES), zero_body, 0)

    @pl.when(has_extra)
    def _():
        pltpu.make_async_copy(pred_hbm.at[pl.ds(base, CHUNK_HI)], pred_v, sem_p).wait()
        pltpu.make_async_copy(target_hbm.at[pl.ds(base, CHUNK_HI)], target_v, sem_t).wait()
        pltpu.make_async_copy(batch_hbm.at[pl.ds(base, CHUNK_HI)], batch_v, sem_b).wait()

    @pl.when(jnp.logical_not(has_extra))
    def _():
        pltpu.make_async_copy(pred_hbm.at[pl.ds(base, CHUNK_LO)],
                              pred_v.at[pl.ds(0, CHUNK_LO)], sem_p).wait()
        pltpu.make_async_copy(target_hbm.at[pl.ds(base, CHUNK_LO)],
                              target_v.at[pl.ds(0, CHUNK_LO)], sem_t).wait()
        pltpu.make_async_copy(batch_hbm.at[pl.ds(base, CHUNK_LO)],
                              batch_v.at[pl.ds(0, CHUNK_LO)], sem_b).wait()

    row_off = lax.iota(jnp.int32, LANES) * ROW_STRIDE

    def scat_one(off):
        p = pred_v[pl.ds(off, LANES)]
        t = target_v[pl.ds(off, LANES)]
        b = batch_v[pl.ds(off, LANES)]
        idx = row_off + b
        plsc.addupdate_scatter(acc, [idx], p * t)
        plsc.addupdate_scatter(acc, [idx + SEG], p + t)

    @plsc.parallel_loop(0, NV_LO * LANES, step=LANES, unroll=5)
    def _(off):
        scat_one(off)

    @pl.when(has_extra)
    def _():
        scat_one(NV_LO * LANES)

    def red_body(cb, carry):
        off = cb * LANES
        v = acc[pl.ds(off, LANES)]
        for r in range(1, LANES):
            v = v + acc[pl.ds(r * ROW_STRIDE + off, LANES)]
        partial_v[pl.ds(off, LANES)] = v
        return carry

    lax.fori_loop(0, ACC_W // LANES, red_body, 0)

    pltpu.sync_copy(partial_v, shared.at[wid])
    plsc.subcore_barrier()

    @pl.when(wid == 0)
    def _():
        pltpu.sync_copy(shared, stage)

        def red2_body(cb, carry):
            off = cb * LANES
            v = stage[0, pl.ds(off, LANES)]
            for r in range(1, LANES):
                v = v + stage[r, pl.ds(off, LANES)]
            partial_v[pl.ds(off, LANES)] = v
            return carry

        lax.fori_loop(0, ACC_W // LANES, red2_body, 0)

        def dice_body(j, s_acc):
            off = j * LANES
            iv = partial_v[pl.ds(off, LANES)]
            dv = partial_v[pl.ds(SEG + off, LANES)]
            return s_acc + (2.0 * iv + 1.0) / (dv + 1.0)

        dice_sum = lax.fori_loop(0, SEG // LANES, dice_body,
                                 jnp.zeros((LANES,), jnp.float32))
        total = jnp.sum(dice_sum)
        out_v[...] = jnp.broadcast_to(float(SEG) - total, (LANES,))
        pltpu.sync_copy(out_v, out_hbm)


_dice_sc = pl.kernel(
    _dice_body,
    out_type=jax.ShapeDtypeStruct((LANES,), jnp.float32),
    mesh=plsc.VectorSubcoreMesh(core_axis_name="c", subcore_axis_name="s",
                                num_cores=1, num_subcores=NS),
    scratch_types=[
        pltpu.VMEM((CHUNK_HI,), jnp.float32),
        pltpu.VMEM((CHUNK_HI,), jnp.float32),
        pltpu.VMEM((CHUNK_HI,), jnp.int32),
        pltpu.VMEM((ACC_FLAT,), jnp.float32),
        pltpu.VMEM((ACC_W,), jnp.float32),
        pltpu.VMEM((LANES, ACC_W), jnp.float32),
        pltpu.VMEM((LANES,), jnp.float32),
        pltpu.VMEM_SHARED((LANES, ACC_W), jnp.float32),
        pltpu.SemaphoreType.DMA,
        pltpu.SemaphoreType.DMA,
        pltpu.SemaphoreType.DMA,
    ],
    compiler_params=pltpu.CompilerParams(needs_layout_passes=False),
)


def kernel(pred, target, batch):
    return _dice_sc(pred, target, batch.astype(jnp.int32))[0]


# speedup vs baseline: 1.1208x; 1.0497x over previous
import jax
import jax.numpy as jnp
from jax import lax
from jax.experimental import pallas as pl
from jax.experimental.pallas import tpu as pltpu
from jax.experimental.pallas import tpu_sc as plsc

N = 100000
SEG = 512
LANES = 16
NS = 16
NV_TOTAL = N // LANES
NV_LO = NV_TOTAL // NS
EXTRA = NV_TOTAL - NV_LO * NS
CHUNK_LO = NV_LO * LANES
CHUNK_HI = (NV_LO + 1) * LANES
ACC_W = 2 * SEG
ROW_STRIDE = ACC_W + 1
ACC_FLAT = 65 * 256


def _dice_body(pred_hbm, target_hbm, batch_hbm, out_hbm,
               pred_v, target_v, batch_v, acc, partial_v,
               stage_a, red_v, iv_v, dv_v, fin_v, out_v,
               shared, shared_red, shared2, sem_p, sem_t, sem_b):
    wid = lax.axis_index("s")
    has_extra = wid < EXTRA
    base = wid * CHUNK_LO + jnp.minimum(wid, EXTRA) * LANES
    nv = NV_LO + has_extra.astype(jnp.int32)

    @pl.when(has_extra)
    def _():
        pltpu.async_copy(pred_hbm.at[pl.ds(base, CHUNK_HI)], pred_v, sem_p)
        pltpu.async_copy(target_hbm.at[pl.ds(base, CHUNK_HI)], target_v, sem_t)
        pltpu.async_copy(batch_hbm.at[pl.ds(base, CHUNK_HI)], batch_v, sem_b)

    @pl.when(jnp.logical_not(has_extra))
    def _():
        pltpu.async_copy(pred_hbm.at[pl.ds(base, CHUNK_LO)],
                         pred_v.at[pl.ds(0, CHUNK_LO)], sem_p)
        pltpu.async_copy(target_hbm.at[pl.ds(base, CHUNK_LO)],
                         target_v.at[pl.ds(0, CHUNK_LO)], sem_t)
        pltpu.async_copy(batch_hbm.at[pl.ds(base, CHUNK_LO)],
                         batch_v.at[pl.ds(0, CHUNK_LO)], sem_b)

    zero = jnp.zeros((LANES,), jnp.float32)

    def zero_body(cb, carry):
        off = cb * (16 * LANES)
        for r in range(16):
            acc[pl.ds(off + r * LANES, LANES)] = zero
        return carry

    lax.fori_loop(0, ACC_FLAT // (16 * LANES), zero_body, 0)

    @pl.when(has_extra)
    def _():
        pltpu.make_async_copy(pred_hbm.at[pl.ds(base, CHUNK_HI)], pred_v, sem_p).wait()
        pltpu.make_async_copy(target_hbm.at[pl.ds(base, CHUNK_HI)], target_v, sem_t).wait()
        pltpu.make_async_copy(batch_hbm.at[pl.ds(base, CHUNK_HI)], batch_v, sem_b).wait()

    @pl.when(jnp.logical_not(has_extra))
    def _():
        pltpu.make_async_copy(pred_hbm.at[pl.ds(base, CHUNK_LO)],
                              pred_v.at[pl.ds(0, CHUNK_LO)], sem_p).wait()
        pltpu.make_async_copy(target_hbm.at[pl.ds(base, CHUNK_LO)],
                              target_v.at[pl.ds(0, CHUNK_LO)], sem_t).wait()
        pltpu.make_async_copy(batch_hbm.at[pl.ds(base, CHUNK_LO)],
                              batch_v.at[pl.ds(0, CHUNK_LO)], sem_b).wait()

    row_off = lax.iota(jnp.int32, LANES) * ROW_STRIDE

    def scat_one(off):
        p = pred_v[pl.ds(off, LANES)]
        t = target_v[pl.ds(off, LANES)]
        b = batch_v[pl.ds(off, LANES)]
        idx = row_off + b
        plsc.addupdate_scatter(acc, [idx], p * t)
        plsc.addupdate_scatter(acc, [idx + SEG], p + t)

    @plsc.parallel_loop(0, NV_LO * LANES, step=LANES, unroll=5)
    def _(off):
        scat_one(off)

    @pl.when(has_extra)
    def _():
        scat_one(NV_LO * LANES)

    @plsc.parallel_loop(0, ACC_W, step=LANES, unroll=2)
    def _(off):
        v = acc[pl.ds(off, LANES)]
        for r in range(1, LANES):
            v = v + acc[pl.ds(r * ROW_STRIDE + off, LANES)]
        partial_v[pl.ds(off, LANES)] = v

    pltpu.sync_copy(partial_v, shared.at[wid])
    plsc.subcore_barrier()

    @pl.when(wid < 8)
    def _():
        pltpu.sync_copy(shared.at[:, pl.ds(wid * 128, 128)], stage_a)
        for cb in range(128 // LANES):
            off = cb * LANES
            v = stage_a[0, pl.ds(off, LANES)]
            for r in range(1, LANES):
                v = v + stage_a[r, pl.ds(off, LANES)]
            red_v[pl.ds(off, LANES)] = v
        pltpu.sync_copy(red_v, shared_red.at[pl.ds(wid * 128, 128)])

    plsc.subcore_barrier()

    @pl.when(wid < 4)
    def _():
        pltpu.sync_copy(shared_red.at[pl.ds(wid * 128, 128)], iv_v)
        pltpu.sync_copy(shared_red.at[pl.ds(SEG + wid * 128, 128)], dv_v)
        dice_sum = jnp.zeros((LANES,), jnp.float32)
        for cb in range(128 // LANES):
            off = cb * LANES
            iv = iv_v[pl.ds(off, LANES)]
            dv = dv_v[pl.ds(off, LANES)]
            dice_sum = dice_sum + (2.0 * iv + 1.0) / (dv + 1.0)
        out_v[...] = dice_sum
        pltpu.sync_copy(out_v, shared2.at[pl.ds(wid * LANES, LANES)])

    plsc.subcore_barrier()

    @pl.when(wid == 0)
    def _():
        pltpu.sync_copy(shared2, fin_v)
        v = fin_v[pl.ds(0, LANES)]
        for k in range(1, 4):
            v = v + fin_v[pl.ds(k * LANES, LANES)]
        total = jnp.sum(v)
        out_v[...] = jnp.broadcast_to(float(SEG) - total, (LANES,))
        pltpu.sync_copy(out_v, out_hbm)


_dice_sc = pl.kernel(
    _dice_body,
    out_type=jax.ShapeDtypeStruct((LANES,), jnp.float32),
    mesh=plsc.VectorSubcoreMesh(core_axis_name="c", subcore_axis_name="s",
                                num_cores=1, num_subcores=NS),
    scratch_types=[
        pltpu.VMEM((CHUNK_HI,), jnp.float32),
        pltpu.VMEM((CHUNK_HI,), jnp.float32),
        pltpu.VMEM((CHUNK_HI,), jnp.int32),
        pltpu.VMEM((ACC_FLAT,), jnp.float32),
        pltpu.VMEM((ACC_W,), jnp.float32),
        pltpu.VMEM((LANES, 128), jnp.float32),
        pltpu.VMEM((128,), jnp.float32),
        pltpu.VMEM((128,), jnp.float32),
        pltpu.VMEM((128,), jnp.float32),
        pltpu.VMEM((4 * LANES,), jnp.float32),
        pltpu.VMEM((LANES,), jnp.float32),
        pltpu.VMEM_SHARED((LANES, ACC_W), jnp.float32),
        pltpu.VMEM_SHARED((ACC_W,), jnp.float32),
        pltpu.VMEM_SHARED((4 * LANES,), jnp.float32),
        pltpu.SemaphoreType.DMA,
        pltpu.SemaphoreType.DMA,
        pltpu.SemaphoreType.DMA,
    ],
    compiler_params=pltpu.CompilerParams(needs_layout_passes=False),
)


def kernel(pred, target, batch):
    return _dice_sc(pred, target, batch.astype(jnp.int32))[0]
